# baseline (speedup 1.0000x reference)
import functools

import jax
import jax.numpy as jnp
import numpy as np
from jax import lax
from jax.experimental import pallas as pl
from jax.experimental.pallas import tpu as pltpu
from jax.experimental.pallas import tpu_sc as plsc

_CLASS_TOKENS = (11, 257, 1024, 4096, 9999, 20000, 50000, 99999)
_NUM_CLASSES = len(_CLASS_TOKENS)
_L = 16
_LN2 = float(np.log(2.0))


_GATHER_DNUMS = lax.GatherDimensionNumbers(
    offset_dims=(), collapsed_slice_dims=(0,), start_index_map=(0,))


def _shuffle(v, perm):
    return lax.gather(v, perm[:, None], _GATHER_DNUMS, slice_sizes=(1,),
                      mode=lax.GatherScatterMode.PROMISE_IN_BOUNDS)


def _all_lanes_max(v, lane):
    for k in (8, 4, 2, 1):
        v = jnp.maximum(v, _shuffle(v, lane ^ k))
    return v


def _all_lanes_sum(v, lane):
    for k in (8, 4, 2, 1):
        v = v + _shuffle(v, lane ^ k)
    return v


@functools.lru_cache(maxsize=None)
def _build_sc_call(B, S, V):
    assert B == 32
    tok16 = _CLASS_TOKENS * (2 * _L // (2 * _NUM_CLASSES))
    rows = np.array(
        [b * S * V + (S - 1) * V + t for b in range(B) for t in tok16],
        dtype=np.int32,
    )

    mesh = plsc.VectorSubcoreMesh(core_axis_name="c", subcore_axis_name="s")
    info = plsc.get_sparse_core_info()
    nc = info.num_cores

    @functools.partial(
        pl.kernel,
        mesh=mesh,
        out_type=jax.ShapeDtypeStruct((B, _L), jnp.float32),
        scratch_types=[
            pltpu.VMEM((_L,), jnp.int32),
            pltpu.VMEM((_L,), jnp.float32),
            pltpu.VMEM((B,), jnp.int32),
            pltpu.VMEM((_L,), jnp.float32),
            pltpu.SemaphoreType.DMA,
        ],
    )
    def sc_call(table_hbm, idx_hbm, tgt_hbm, out_hbm,
                idx_v, vals_v, tgt_v, out_v, sem):
        wid = lax.axis_index("s") * nc + lax.axis_index("c")
        pltpu.sync_copy(idx_hbm.at[pl.ds(wid * _L, _L)], idx_v)
        pltpu.sync_copy(tgt_hbm, tgt_v)
        pltpu.async_copy(table_hbm.at[idx_v], vals_v, sem).wait()

        lane = lax.iota(jnp.int32, _L)
        vals = vals_v[...]

        m = _all_lanes_max(vals, lane)
        s = _all_lanes_sum(jnp.exp(vals - m), lane) * 0.5
        bits = lax.bitcast_convert_type(s, jnp.int32)
        e = ((bits >> 23) - 127).astype(jnp.float32)
        f = lax.bitcast_convert_type((bits & 0x007FFFFF) | 0x3F800000,
                                     jnp.float32)
        r = (f - 1.0) / (f + 1.0)
        r2 = r * r
        log_f = 2.0 * r * (1.0 + r2 * (1.0 / 3.0 + r2 * (
            1.0 / 5.0 + r2 * (1.0 / 9.0 * r2 + 1.0 / 7.0))))
        lse_v = m + e * _LN2 + log_f

        widv = jnp.full((_L,), wid, dtype=jnp.int32)
        widm = widv & (_L - 1)
        tgt_splat = jnp.where(widv < _L,
                              _shuffle(tgt_v[pl.ds(0, _L)], widm),
                              _shuffle(tgt_v[pl.ds(_L, _L)], widm))
        picked = _shuffle(vals, tgt_splat)

        out_v[...] = lse_v - picked
        pltpu.sync_copy(out_v, out_hbm.at[wid])

    def run(model_outputs, targets):
        table = model_outputs.reshape(B * S * V)
        out2d = sc_call(table, jnp.asarray(rows),
                        targets.reshape(B).astype(jnp.int32))
        return out2d[:, 0]

    return run


def kernel(model_outputs, targets, input_pos):
    B, S, V = model_outputs.shape
    return _build_sc_call(B, S, V)(model_outputs, targets)


# speedup vs baseline: 7.4074x; 7.4074x over previous
import functools

import jax
import jax.numpy as jnp
import numpy as np
from jax import lax
from jax.experimental import pallas as pl
from jax.experimental.pallas import tpu as pltpu
from jax.experimental.pallas import tpu_sc as plsc

_CLASS_TOKENS = (11, 257, 1024, 4096, 9999, 20000, 50000, 99999)
_NUM_CLASSES = len(_CLASS_TOKENS)
_L = 16
_LN2 = float(np.log(2.0))

_GATHER_DNUMS = lax.GatherDimensionNumbers(
    offset_dims=(), collapsed_slice_dims=(0,), start_index_map=(0,))


def _shuffle(v, perm):
    return lax.gather(v, perm[:, None], _GATHER_DNUMS, slice_sizes=(1,),
                      mode=lax.GatherScatterMode.PROMISE_IN_BOUNDS)


@functools.lru_cache(maxsize=None)
def _build_sc_call(B, S, V):
    assert B == 32 and V > max(_CLASS_TOKENS)
    bases = [t & ~(_L - 1) for t in _CLASS_TOKENS]
    lanes = [t & (_L - 1) for t in _CLASS_TOKENS]

    mesh = plsc.VectorSubcoreMesh(core_axis_name="c", subcore_axis_name="s")
    info = plsc.get_sparse_core_info()
    nc = info.num_cores

    @functools.partial(
        pl.kernel,
        mesh=mesh,
        out_type=jax.ShapeDtypeStruct((B, _L), jnp.float32),
        scratch_types=(
            [pltpu.VMEM((_L,), jnp.float32) for _ in range(_NUM_CLASSES)]
            + [
                pltpu.VMEM((B,), jnp.int32),
                pltpu.VMEM((_L,), jnp.float32),
                pltpu.SemaphoreType.DMA,
            ]
        ),
    )
    def sc_call(mo_hbm, tgt_hbm, out_hbm, *rest):
        bufs, (tgt_v, out_v, sem) = rest[:_NUM_CLASSES], rest[_NUM_CLASSES:]
        wid = lax.axis_index("s") * nc + lax.axis_index("c")
        copies = [
            pltpu.async_copy(mo_hbm.at[wid, S - 1, pl.ds(bases[j], _L)],
                             bufs[j], sem)
            for j in range(_NUM_CLASSES)
        ]
        pltpu.sync_copy(tgt_hbm, tgt_v)
        for c in copies:
            c.wait()

        picks = [
            _shuffle(bufs[j][...], jnp.full((_L,), lanes[j], jnp.int32))
            for j in range(_NUM_CLASSES)
        ]
        m = picks[0]
        for p in picks[1:]:
            m = jnp.maximum(m, p)
        s = jnp.exp(picks[0] - m)
        for p in picks[1:]:
            s = s + jnp.exp(p - m)
        bits = lax.bitcast_convert_type(s, jnp.int32)
        e = ((bits >> 23) - 127).astype(jnp.float32)
        f = lax.bitcast_convert_type((bits & 0x007FFFFF) | 0x3F800000,
                                     jnp.float32)
        r = (f - 1.0) / (f + 1.0)
        r2 = r * r
        log_f = 2.0 * r * (1.0 + r2 * (1.0 / 3.0 + r2 * (
            1.0 / 5.0 + r2 * (1.0 / 9.0 * r2 + 1.0 / 7.0))))
        lse_v = m + e * _LN2 + log_f

        widv = jnp.full((_L,), wid, dtype=jnp.int32)
        widm = widv & (_L - 1)
        tgt_splat = jnp.where(widv < _L,
                              _shuffle(tgt_v[pl.ds(0, _L)], widm),
                              _shuffle(tgt_v[pl.ds(_L, _L)], widm))
        picked = picks[0]
        for j in range(1, _NUM_CLASSES):
            picked = jnp.where(tgt_splat == j, picks[j], picked)

        out_v[...] = lse_v - picked
        pltpu.sync_copy(out_v, out_hbm.at[wid])

    def run(model_outputs, targets):
        out2d = sc_call(model_outputs, targets.reshape(B).astype(jnp.int32))
        return out2d[:, 0]

    return run


def kernel(model_outputs, targets, input_pos):
    B, S, V = model_outputs.shape
    return _build_sc_call(B, S, V)(model_outputs, targets)
